# baseline (speedup 1.0000x reference)
import functools

import jax
import jax.numpy as jnp
from jax import lax
from jax.experimental import pallas as pl
from jax.experimental.pallas import tpu as pltpu
from jax.experimental.pallas import tpu_sc as plsc

N_NODES = 10000
N_EDGES = 320000
D = 128

NC = 2
NS = 16
CHUNK = 128
C0 = 144
C1 = 24
PHASES = 3
PC0 = C0 // PHASES
PC1 = C1 // PHASES
SC0_ROWS = NS * C0
N_ROWS = NS * (C0 + C1)
N_ROWS_PAD = 2736
EDGES_PAD = N_ROWS * CHUNK
N_NODES_PAD = 10112
ROWS_PER_TILE = N_NODES_PAD // NS


def _sc_gather_scatter(feat2, src2, dst2, zeros):
    mesh = plsc.VectorSubcoreMesh(core_axis_name="c", subcore_axis_name="s")

    @functools.partial(
        pl.kernel,
        out_type=jax.ShapeDtypeStruct((NC, N_NODES_PAD, D), jnp.float32),
        mesh=mesh,
        scratch_types=[
            pltpu.VMEM((PC0, CHUNK), jnp.int32),
            pltpu.VMEM((PC0, CHUNK), jnp.int32),
            pltpu.VMEM((CHUNK, D), jnp.float32),
            pltpu.VMEM((CHUNK, D), jnp.float32),
            pltpu.VMEM_SHARED((N_NODES_PAD, D), jnp.float32),
            pltpu.SemaphoreType.DMA,
            pltpu.SemaphoreType.DMA,
            pltpu.SemaphoreType.DMA,
            pltpu.SemaphoreType.DMA,
        ],
    )
    def k(feat_hbm, src_hbm, dst_hbm, zeros_hbm, out_hbm,
          src_v, dst_v, rows0, rows1, acc, g0, g1, s0, s1):
        rows = (rows0, rows1)
        gsems = (g0, g1)
        ssems = (s0, s1)
        c = lax.axis_index("c")
        s = lax.axis_index("s")
        table = feat_hbm.at[c]
        pc = jnp.where(c == 0, PC0, PC1)
        base = jnp.where(c == 0, s * C0, SC0_ROWS + s * C1)

        r0 = s * ROWS_PER_TILE
        pltpu.sync_copy(zeros_hbm.at[pl.ds(r0, ROWS_PER_TILE)],
                        acc.at[pl.ds(r0, ROWS_PER_TILE)])
        plsc.subcore_barrier()

        def start_gather(t, b):
            pltpu.async_copy(table.at[src_v.at[t]], rows[b], gsems[b])

        def wait_gather(t, b):
            pltpu.make_async_copy(
                table.at[src_v.at[t]], rows[b], gsems[b]).wait()

        def start_scatter(t, b):
            pltpu.async_copy(rows[b], acc.at[dst_v.at[t]], ssems[b], add=True)

        def wait_scatter(t, b):
            pltpu.make_async_copy(
                rows[b], acc.at[dst_v.at[t]], ssems[b]).wait()

        for p in range(PHASES):
            off = base + p * pc
            pltpu.sync_copy(src_hbm.at[pl.ds(off, PC0)], src_v)
            pltpu.sync_copy(dst_hbm.at[pl.ds(off, PC0)], dst_v)

            start_gather(0, 0)
            wait_gather(0, 0); start_scatter(0, 0); start_gather(1, 1)

            def epoch(e, carry):
                t0 = 2 * e + 1
                for i in range(2):
                    t = t0 + i
                    b = (1 + i) % 2
                    wait_gather(t, b)
                    wait_scatter(t - 1, 1 - b)
                    start_gather(t + 1, 1 - b)
                    start_scatter(t, b)
                return carry

            lax.fori_loop(0, (pc - 2) // 2, epoch, 0)

            t = pc - 1
            wait_gather(t, 1); wait_scatter(t - 1, 0); start_scatter(t, 1)
            wait_scatter(t, 1)

        plsc.subcore_barrier()
        pltpu.sync_copy(acc.at[pl.ds(r0, ROWS_PER_TILE)],
                        out_hbm.at[c, pl.ds(r0, ROWS_PER_TILE)])

    return k(feat2, src2, dst2, zeros)


def _tc_combine(feat, partial):
    blk = 1000

    def body(f_ref, p0_ref, p1_ref, o_ref):
        o_ref[...] = f_ref[...] + p0_ref[0] + p1_ref[0]

    return pl.pallas_call(
        body,
        grid=(N_NODES // blk,),
        in_specs=[
            pl.BlockSpec((blk, D), lambda i: (i, 0)),
            pl.BlockSpec((1, blk, D), lambda i: (0, i, 0)),
            pl.BlockSpec((1, blk, D), lambda i: (1, i, 0)),
        ],
        out_specs=pl.BlockSpec((blk, D), lambda i: (i, 0)),
        out_shape=jax.ShapeDtypeStruct((N_NODES, D), jnp.float32),
    )(feat, partial, partial)


@jax.jit
def kernel(feat, edge_index):
    ei = edge_index.astype(jnp.int32)
    pad = EDGES_PAD - N_EDGES
    trash = N_NODES + jnp.arange(pad, dtype=jnp.int32) % (N_NODES_PAD - N_NODES)
    src = jnp.concatenate([ei[0], jnp.zeros((pad,), jnp.int32)])
    dst = jnp.concatenate([ei[1], trash])
    over = (N_ROWS_PAD - N_ROWS) * CHUNK
    src2 = jnp.concatenate([src, jnp.zeros((over,), jnp.int32)])
    dst2 = jnp.concatenate([dst, jnp.zeros((over,), jnp.int32)])
    src2 = src2.reshape(N_ROWS_PAD, CHUNK)
    dst2 = dst2.reshape(N_ROWS_PAD, CHUNK)
    zeros = jnp.zeros((N_NODES_PAD, D), jnp.float32)
    feat2 = jnp.stack([feat, feat])
    partial = _sc_gather_scatter(feat2, src2, dst2, zeros)
    return _tc_combine(feat, partial)


# speedup vs baseline: 2.3886x; 2.3886x over previous
import functools

import jax
import jax.numpy as jnp
from jax import lax
from jax.experimental import pallas as pl
from jax.experimental.pallas import tpu as pltpu
from jax.experimental.pallas import tpu_sc as plsc

N_NODES = 10000
N_EDGES = 320000
D = 128

NC = 2
NS = 16
CHUNK = 128
C0 = 128
C1 = 32
PHASES = 4
PC0 = C0 // PHASES
PC1 = C1 // PHASES
SC0_ROWS = NS * C0
N_ROWS = NS * (C0 + C1)
N_ROWS_PAD = 2592
EDGES_PAD = N_ROWS * CHUNK
N_NODES_PAD = 10112
ROWS_PER_TILE = N_NODES_PAD // NS


def _sc_gather_scatter(feat2, src2, dst2, zeros):
    mesh = plsc.VectorSubcoreMesh(core_axis_name="c", subcore_axis_name="s")

    @functools.partial(
        pl.kernel,
        out_type=jax.ShapeDtypeStruct((NC, N_NODES_PAD, D), jnp.float32),
        mesh=mesh,
        scratch_types=[
            pltpu.VMEM((PC0, CHUNK), jnp.int32),
            pltpu.VMEM((PC0, CHUNK), jnp.int32),
            pltpu.VMEM((CHUNK, D), jnp.float32),
            pltpu.VMEM((CHUNK, D), jnp.float32),
            pltpu.VMEM_SHARED((N_NODES_PAD, D), jnp.float32),
            pltpu.SemaphoreType.DMA,
            pltpu.SemaphoreType.DMA,
            pltpu.SemaphoreType.DMA,
            pltpu.SemaphoreType.DMA,
        ],
    )
    def k(feat_hbm, src_hbm, dst_hbm, zeros_hbm, out_hbm,
          src_v, dst_v, rows0, rows1, acc, g0, g1, s0, s1):
        rows = (rows0, rows1)
        gsems = (g0, g1)
        ssems = (s0, s1)
        c = lax.axis_index("c")
        s = lax.axis_index("s")
        table = feat_hbm.at[c]
        pc = jnp.where(c == 0, PC0, PC1)
        base = jnp.where(c == 0, s * C0, SC0_ROWS + s * C1)

        r0 = s * ROWS_PER_TILE
        pltpu.sync_copy(zeros_hbm.at[pl.ds(r0, ROWS_PER_TILE)],
                        acc.at[pl.ds(r0, ROWS_PER_TILE)])
        plsc.subcore_barrier()

        def start_gather(t, b):
            pltpu.async_copy(table.at[src_v.at[t]], rows[b], gsems[b])

        def wait_gather(t, b):
            pltpu.make_async_copy(
                table.at[src_v.at[t]], rows[b], gsems[b]).wait()

        def start_scatter(t, b):
            pltpu.async_copy(rows[b], acc.at[dst_v.at[t]], ssems[b], add=True)

        def wait_scatter(t, b):
            pltpu.make_async_copy(
                rows[b], acc.at[dst_v.at[t]], ssems[b]).wait()

        for p in range(PHASES):
            off = base + p * pc
            pltpu.sync_copy(src_hbm.at[pl.ds(off, PC0)], src_v)
            pltpu.sync_copy(dst_hbm.at[pl.ds(off, PC0)], dst_v)

            start_gather(0, 0)
            wait_gather(0, 0); start_scatter(0, 0); start_gather(1, 1)

            def epoch(e, carry):
                t0 = 2 * e + 1
                for i in range(2):
                    t = t0 + i
                    b = (1 + i) % 2
                    wait_gather(t, b)
                    wait_scatter(t - 1, 1 - b)
                    start_gather(t + 1, 1 - b)
                    start_scatter(t, b)
                return carry

            lax.fori_loop(0, (pc - 2) // 2, epoch, 0)

            t = pc - 1
            wait_gather(t, 1); wait_scatter(t - 1, 0); start_scatter(t, 1)
            wait_scatter(t, 1)

        plsc.subcore_barrier()
        pltpu.sync_copy(acc.at[pl.ds(r0, ROWS_PER_TILE)],
                        out_hbm.at[c, pl.ds(r0, ROWS_PER_TILE)])

    return k(feat2, src2, dst2, zeros)


def _tc_combine(feat, partial):
    blk = 1000

    def body(f_ref, p0_ref, p1_ref, o_ref):
        o_ref[...] = f_ref[...] + p0_ref[0] + p1_ref[0]

    return pl.pallas_call(
        body,
        grid=(N_NODES // blk,),
        in_specs=[
            pl.BlockSpec((blk, D), lambda i: (i, 0)),
            pl.BlockSpec((1, blk, D), lambda i: (0, i, 0)),
            pl.BlockSpec((1, blk, D), lambda i: (1, i, 0)),
        ],
        out_specs=pl.BlockSpec((blk, D), lambda i: (i, 0)),
        out_shape=jax.ShapeDtypeStruct((N_NODES, D), jnp.float32),
    )(feat, partial, partial)


@jax.jit
def kernel(feat, edge_index):
    ei = edge_index.astype(jnp.int32)
    pad = EDGES_PAD - N_EDGES
    trash = N_NODES + jnp.arange(pad, dtype=jnp.int32) % (N_NODES_PAD - N_NODES)
    src = jnp.concatenate([ei[0], jnp.zeros((pad,), jnp.int32)])
    dst = jnp.concatenate([ei[1], trash])
    over = (N_ROWS_PAD - N_ROWS) * CHUNK
    src2 = jnp.concatenate([src, jnp.zeros((over,), jnp.int32)])
    dst2 = jnp.concatenate([dst, jnp.zeros((over,), jnp.int32)])
    src2 = src2.reshape(N_ROWS_PAD, CHUNK)
    dst2 = dst2.reshape(N_ROWS_PAD, CHUNK)
    zeros = jnp.zeros((N_NODES_PAD, D), jnp.float32)
    feat2 = jnp.stack([feat, feat])
    partial = _sc_gather_scatter(feat2, src2, dst2, zeros)
    return _tc_combine(feat, partial)


# speedup vs baseline: 2.3923x; 1.0015x over previous
import functools

import jax
import jax.numpy as jnp
from jax import lax
from jax.experimental import pallas as pl
from jax.experimental.pallas import tpu as pltpu
from jax.experimental.pallas import tpu_sc as plsc

N_NODES = 10000
N_EDGES = 320000
D = 128

NC = 2
NS = 16
CHUNK = 128
C0 = 128
C1 = 32
PHASES = 2
PC0 = C0 // PHASES
PC1 = C1 // PHASES
SC0_ROWS = NS * C0
N_ROWS = NS * (C0 + C1)
N_ROWS_PAD = 2608
EDGES_PAD = N_ROWS * CHUNK
N_NODES_PAD = 10112
ROWS_PER_TILE = N_NODES_PAD // NS


def _sc_gather_scatter(feat2, src2, dst2, zeros):
    mesh = plsc.VectorSubcoreMesh(core_axis_name="c", subcore_axis_name="s")

    @functools.partial(
        pl.kernel,
        out_type=jax.ShapeDtypeStruct((NC, N_NODES_PAD, D), jnp.float32),
        mesh=mesh,
        scratch_types=[
            pltpu.VMEM((PC0, CHUNK), jnp.int32),
            pltpu.VMEM((PC0, CHUNK), jnp.int32),
            pltpu.VMEM((CHUNK, D), jnp.float32),
            pltpu.VMEM((CHUNK, D), jnp.float32),
            pltpu.VMEM_SHARED((N_NODES_PAD, D), jnp.float32),
            pltpu.SemaphoreType.DMA,
            pltpu.SemaphoreType.DMA,
            pltpu.SemaphoreType.DMA,
            pltpu.SemaphoreType.DMA,
        ],
    )
    def k(feat_hbm, src_hbm, dst_hbm, zeros_hbm, out_hbm,
          src_v, dst_v, rows0, rows1, acc, g0, g1, s0, s1):
        rows = (rows0, rows1)
        gsems = (g0, g1)
        ssems = (s0, s1)
        c = lax.axis_index("c")
        s = lax.axis_index("s")
        table = feat_hbm.at[c]
        pc = jnp.where(c == 0, PC0, PC1)
        base = jnp.where(c == 0, s * C0, SC0_ROWS + s * C1)

        r0 = s * ROWS_PER_TILE
        pltpu.sync_copy(zeros_hbm.at[pl.ds(r0, ROWS_PER_TILE)],
                        acc.at[pl.ds(r0, ROWS_PER_TILE)])
        plsc.subcore_barrier()

        def start_gather(t, b):
            pltpu.async_copy(table.at[src_v.at[t]], rows[b], gsems[b])

        def wait_gather(t, b):
            pltpu.make_async_copy(
                table.at[src_v.at[t]], rows[b], gsems[b]).wait()

        def start_scatter(t, b):
            pltpu.async_copy(rows[b], acc.at[dst_v.at[t]], ssems[b], add=True)

        def wait_scatter(t, b):
            pltpu.make_async_copy(
                rows[b], acc.at[dst_v.at[t]], ssems[b]).wait()

        for p in range(PHASES):
            off = base + p * pc
            pltpu.sync_copy(src_hbm.at[pl.ds(off, PC0)], src_v)
            pltpu.sync_copy(dst_hbm.at[pl.ds(off, PC0)], dst_v)

            start_gather(0, 0)
            wait_gather(0, 0); start_scatter(0, 0); start_gather(1, 1)

            def epoch(e, carry):
                t0 = 2 * e + 1
                for i in range(2):
                    t = t0 + i
                    b = (1 + i) % 2
                    wait_gather(t, b)
                    wait_scatter(t - 1, 1 - b)
                    start_gather(t + 1, 1 - b)
                    start_scatter(t, b)
                return carry

            lax.fori_loop(0, (pc - 2) // 2, epoch, 0)

            t = pc - 1
            wait_gather(t, 1); wait_scatter(t - 1, 0); start_scatter(t, 1)
            wait_scatter(t, 1)

        plsc.subcore_barrier()
        pltpu.sync_copy(acc.at[pl.ds(r0, ROWS_PER_TILE)],
                        out_hbm.at[c, pl.ds(r0, ROWS_PER_TILE)])

    return k(feat2, src2, dst2, zeros)


def _tc_combine(feat, partial):
    blk = 1000

    def body(f_ref, p0_ref, p1_ref, o_ref):
        o_ref[...] = f_ref[...] + p0_ref[0] + p1_ref[0]

    return pl.pallas_call(
        body,
        grid=(N_NODES // blk,),
        in_specs=[
            pl.BlockSpec((blk, D), lambda i: (i, 0)),
            pl.BlockSpec((1, blk, D), lambda i: (0, i, 0)),
            pl.BlockSpec((1, blk, D), lambda i: (1, i, 0)),
        ],
        out_specs=pl.BlockSpec((blk, D), lambda i: (i, 0)),
        out_shape=jax.ShapeDtypeStruct((N_NODES, D), jnp.float32),
    )(feat, partial, partial)


@jax.jit
def kernel(feat, edge_index):
    ei = edge_index.astype(jnp.int32)
    pad = EDGES_PAD - N_EDGES
    trash = N_NODES + jnp.arange(pad, dtype=jnp.int32) % (N_NODES_PAD - N_NODES)
    src = jnp.concatenate([ei[0], jnp.zeros((pad,), jnp.int32)])
    dst = jnp.concatenate([ei[1], trash])
    over = (N_ROWS_PAD - N_ROWS) * CHUNK
    src2 = jnp.concatenate([src, jnp.zeros((over,), jnp.int32)])
    dst2 = jnp.concatenate([dst, jnp.zeros((over,), jnp.int32)])
    src2 = src2.reshape(N_ROWS_PAD, CHUNK)
    dst2 = dst2.reshape(N_ROWS_PAD, CHUNK)
    zeros = jnp.zeros((N_NODES_PAD, D), jnp.float32)
    feat2 = jnp.stack([feat, feat])
    partial = _sc_gather_scatter(feat2, src2, dst2, zeros)
    return _tc_combine(feat, partial)
